# baseline (speedup 1.0000x reference)
import functools

import jax
import jax.numpy as jnp
from jax import lax
from jax.experimental import pallas as pl
from jax.experimental.pallas import tpu as pltpu
from jax.experimental.pallas import tpu_sc as plsc

N = 10000
E = 320000
D = 128
H = 64
G = 64
EPS = 1e-5

NC = 2
NS = 16
NW = NC * NS
ROWS = 320
NP = NW * ROWS
NPT = NP + 8
SC_C = 128
BIN_CH = 2000
EW = E // NW
PCAP = 79 * 128
SEGC = 160
H2 = 128
NPAIR = NW * NW
EPAD = 220 * 2048
MLP_BLK = 2048

_mesh = plsc.VectorSubcoreMesh(
    core_axis_name="c", subcore_axis_name="s", num_cores=NC, num_subcores=NS
)


def _wid():
    return lax.axis_index("s") * NC + lax.axis_index("c")


def _base_offset_v(countsv, wid):
    def body(i, off):
        c = countsv[pl.ds(pl.multiple_of(i * 16, 16), 16)][0]
        p = ((c + (SC_C - 1)) // SC_C) * SC_C
        return off + jnp.where(i < wid * NW, p, 0)
    return lax.fori_loop(0, NPAIR, body, jnp.int32(0))


def _bin_body(dst_hbm, src_hbm, dlist_hbm, slist_hbm, counts_hbm,
              dbuf, sbuf, dstage, sstage, cursv, flv, cbuf):
    wid = _wid()

    def init16(i, c):
        cursv[pl.ds(i * 16, 16)] = jnp.zeros((16,), jnp.int32)
        flv[pl.ds(i * 16, 16)] = jnp.zeros((16,), jnp.int32)
        return c
    lax.fori_loop(0, NW, init16, 0)

    def inits(i, c):
        dstage[pl.ds(i * 16, 16)] = jnp.full((16,), ROWS, jnp.int32)
        sstage[pl.ds(i * 16, 16)] = jnp.zeros((16,), jnp.int32)
        return c
    lax.fori_loop(0, (NW * SEGC) // 16, inits, 0)

    ebase = wid * EW

    def chunk(ci, c):
        pltpu.sync_copy(dst_hbm.at[pl.ds(ebase + ci * BIN_CH, BIN_CH)], dbuf)
        pltpu.sync_copy(src_hbm.at[pl.ds(ebase + ci * BIN_CH, BIN_CH)], sbuf)

        def vreg(j, c2):
            d16 = dbuf[pl.ds(j * 16, 16)]
            s16 = sbuf[pl.ds(j * 16, 16)]
            for j2 in range(16):
                d = d16[j2]
                s = s16[j2]
                o = (d * 6554) >> 21
                ob = pl.multiple_of(o * 16, 16)
                cur = cursv[pl.ds(ob, 16)][0]
                pos = o * SEGC + cur
                dstage[pl.ds(pos, 16)] = jnp.broadcast_to(d - o * ROWS, (16,))
                sstage[pl.ds(pos, 16)] = jnp.broadcast_to(s, (16,))
                cur = cur + 1

                def do_flush(cur2):
                    fl = flv[pl.ds(ob, 16)][0]
                    pb = pl.multiple_of(
                        (o * NW + wid) * PCAP + fl * SC_C, SC_C)
                    sb = pl.multiple_of(o * SEGC, 16)
                    pltpu.sync_copy(dstage.at[pl.ds(sb, SC_C)],
                                    dlist_hbm.at[pl.ds(pb, SC_C)])
                    pltpu.sync_copy(sstage.at[pl.ds(sb, SC_C)],
                                    slist_hbm.at[pl.ds(pb, SC_C)])
                    flv[pl.ds(ob, 16)] = jnp.broadcast_to(fl + 1, (16,))
                    return cur2 - SC_C

                cur = lax.cond(cur >= SC_C, do_flush, lambda x: x, cur)
                cursv[pl.ds(ob, 16)] = jnp.broadcast_to(cur, (16,))
            return c2
        lax.fori_loop(0, BIN_CH // 16, vreg, 0)
        return c

    lax.fori_loop(0, EW // BIN_CH, chunk, 0)

    def fin(o, c):
        ob = pl.multiple_of(o * 16, 16)
        cur = cursv[pl.ds(ob, 16)][0]
        fl = flv[pl.ds(ob, 16)][0]

        def do_f(x):
            pb = pl.multiple_of((o * NW + wid) * PCAP + fl * SC_C, SC_C)
            sb = pl.multiple_of(o * SEGC, 16)
            pltpu.sync_copy(dstage.at[pl.ds(sb, SC_C)],
                            dlist_hbm.at[pl.ds(pb, SC_C)])
            pltpu.sync_copy(sstage.at[pl.ds(sb, SC_C)],
                            slist_hbm.at[pl.ds(pb, SC_C)])
            return x
        lax.cond(cur > 0, do_f, lambda x: x, 0)
        cbuf[pl.ds(o * 16, 16)] = jnp.broadcast_to(fl * SC_C + cur, (16,))
        return c
    lax.fori_loop(0, NW, fin, 0)
    pltpu.sync_copy(
        cbuf, counts_hbm.at[pl.ds(pl.multiple_of(wid * NW * 16, 16),
                                  NW * 16)])


@jax.jit
def _bin_call(dst, src):
    f = pl.kernel(
        _bin_body,
        out_type=[
            jax.ShapeDtypeStruct((NPAIR * PCAP,), jnp.int32),
            jax.ShapeDtypeStruct((NPAIR * PCAP,), jnp.int32),
            jax.ShapeDtypeStruct((NPAIR * 16,), jnp.int32),
        ],
        mesh=_mesh,
        scratch_types=[
            pltpu.VMEM((BIN_CH,), jnp.int32),
            pltpu.VMEM((BIN_CH,), jnp.int32),
            pltpu.VMEM((NW * SEGC,), jnp.int32),
            pltpu.VMEM((NW * SEGC,), jnp.int32),
            pltpu.VMEM((NW * 16,), jnp.int32),
            pltpu.VMEM((NW * 16,), jnp.int32),
            pltpu.VMEM((NW * 16,), jnp.int32),
        ],
        name="sc_bin_edges",
    )
    return f(dst, src)


def _tables_kernel(x_ref, w_ref, b_ref, t_ref, *, F):
    x = x_ref[...]
    wa = w_ref[0:F, :]
    wb = w_ref[F:2 * F, :]
    p = jnp.dot(x, wa - wb, preferred_element_type=jnp.float32,
                precision=lax.Precision.HIGHEST) + b_ref[...]
    q = jnp.dot(x, wb, preferred_element_type=jnp.float32,
                precision=lax.Precision.HIGHEST)
    t_ref[...] = jnp.concatenate([p, q], axis=1)


@jax.jit
def _tables_call(x_pad, w, b):
    F = x_pad.shape[1]
    return pl.pallas_call(
        functools.partial(_tables_kernel, F=F),
        out_shape=jax.ShapeDtypeStruct((NPT, H2), jnp.float32),
    )(x_pad, w, b)


def _mlp_kernel(a_ref, w_ref, bias_ref, o_ref):
    ab = a_ref[...]
    h = jnp.maximum(ab[:, 0:H] + ab[:, H:H2], 0.0)
    h = jnp.dot(h, w_ref[...], preferred_element_type=jnp.float32,
                precision=lax.Precision.HIGHEST) + bias_ref[...]
    h = jnp.maximum(h, 0.0)
    o_ref[...] = jnp.concatenate(
        [h, jnp.zeros((h.shape[0], H), jnp.float32)], axis=1)


@jax.jit
def _mlp_call(a, w, bias):
    grid = EPAD // MLP_BLK
    return pl.pallas_call(
        _mlp_kernel,
        grid=(grid,),
        in_specs=[
            pl.BlockSpec((MLP_BLK, H2), lambda i: (i, 0)),
            pl.BlockSpec((H, H), lambda i: (0, 0)),
            pl.BlockSpec((1, H), lambda i: (0, 0)),
        ],
        out_specs=pl.BlockSpec((MLP_BLK, H2), lambda i: (i, 0)),
        out_shape=jax.ShapeDtypeStruct((EPAD, H2), jnp.float32),
    )(a, w, bias)


def _bn_stats(h):
    mean = jnp.sum(h, axis=0, keepdims=True) / N
    ss = jnp.sum((h - mean) ** 2, axis=0, keepdims=True)
    var = (ss - (NP - N) * mean * mean) / N
    rstd = lax.rsqrt(var + EPS)
    return mean, rstd


def _bn_tables_kernel(h_ref, g_ref, be_ref, w_ref, b_ref, t_ref):
    h = h_ref[...][:, 0:H]
    mean, rstd = _bn_stats(h)
    hbn = g_ref[...] * (h - mean) * rstd + be_ref[...]
    wa = w_ref[0:H, :]
    wb = w_ref[H:2 * H, :]
    p = jnp.dot(hbn, wa - wb, preferred_element_type=jnp.float32,
                precision=lax.Precision.HIGHEST) + b_ref[...]
    q = jnp.dot(hbn, wb, preferred_element_type=jnp.float32,
                precision=lax.Precision.HIGHEST)
    t_ref[0:NP, :] = jnp.concatenate([p, q], axis=1)
    t_ref[NP:NPT, :] = jnp.zeros((NPT - NP, H2), jnp.float32)


@jax.jit
def _bn_tables_call(hraw, gamma, beta, w, b):
    return pl.pallas_call(
        _bn_tables_kernel,
        out_shape=jax.ShapeDtypeStruct((NPT, H2), jnp.float32),
    )(hraw, gamma, beta, w, b)


def _bn_kernel(h_ref, g_ref, be_ref, o_ref, m_ref, r_ref):
    h = h_ref[...][:, 0:H]
    mean, rstd = _bn_stats(h)
    o_ref[...] = g_ref[...] * (h - mean) * rstd + be_ref[...]
    m_ref[...] = mean
    r_ref[...] = rstd


@jax.jit
def _bn_call(hraw, gamma, beta):
    return pl.pallas_call(
        _bn_kernel,
        out_shape=[
            jax.ShapeDtypeStruct((NP, H), jnp.float32),
            jax.ShapeDtypeStruct((1, H), jnp.float32),
            jax.ShapeDtypeStruct((1, H), jnp.float32),
        ],
    )(hraw, gamma, beta)


def _pool_kernel(h_ref, b_ref, m_ref, r_ref, g_ref, be_ref, o_ref):
    g = pl.program_id(0)
    h = h_ref[...][:, 0:H]
    mcol = b_ref[...] == g
    cnt = jnp.sum(mcol.astype(jnp.float32))
    s = jnp.sum(jnp.where(mcol, h, 0.0), axis=0, keepdims=True)
    mx = jnp.max(jnp.where(mcol, h, -3e38), axis=0, keepdims=True)
    mn = jnp.min(jnp.where(mcol, h, 3e38), axis=0, keepdims=True)
    gm = g_ref[...]
    segmean = s / jnp.maximum(cnt, 1.0)
    meancol = gm * (segmean - m_ref[...]) * r_ref[...] + be_ref[...]
    ext = jnp.where(gm >= 0.0, mx, mn)
    maxcol = gm * (ext - m_ref[...]) * r_ref[...] + be_ref[...]
    nonempty = cnt > 0.0
    meancol = jnp.where(nonempty, meancol, 0.0)
    maxcol = jnp.where(nonempty, maxcol, 0.0)
    o_ref[...] = jnp.concatenate([meancol, maxcol], axis=1).reshape(1, 1, 2 * H)


@jax.jit
def _pool_call(hraw, batch2d, mean, rstd, gamma, beta):
    return pl.pallas_call(
        _pool_kernel,
        grid=(G,),
        in_specs=[
            pl.BlockSpec((NP, H2), lambda i: (0, 0)),
            pl.BlockSpec((NP, 1), lambda i: (0, 0)),
            pl.BlockSpec((1, H), lambda i: (0, 0)),
            pl.BlockSpec((1, H), lambda i: (0, 0)),
            pl.BlockSpec((1, H), lambda i: (0, 0)),
            pl.BlockSpec((1, H), lambda i: (0, 0)),
        ],
        out_specs=pl.BlockSpec((1, 1, 2 * H), lambda i: (i, 0, 0)),
        out_shape=jax.ShapeDtypeStruct((G, 1, 2 * H), jnp.float32),
    )(hraw, batch2d, mean, rstd, gamma, beta)


def kernel(x, edge_index, batch, W1, b1, W2, b2, gamma1, beta1,
           W3, b3, W4, b4, gamma2, beta2):
    src = edge_index[0]
    dst = edge_index[1]
    x_pad = jnp.pad(x, ((0, NPT - N), (0, 0)))
    batch2d = jnp.pad(batch, (0, NP - N), constant_values=G)[:, None]
    b1r = b1[None, :]
    b2r = b2[None, :]
    b3r = b3[None, :]
    b4r = b4[None, :]
    g1r = gamma1[None, :]
    be1r = beta1[None, :]
    g2r = gamma2[None, :]
    be2r = beta2[None, :]

    dlist, slist, counts = _bin_call(dst, src)
    keep = jnp.minimum(counts[0].astype(jnp.float32), 0.0)

    def _edge_pre(t):
        return t[:, 0:H][dst] + t[:, H:H2][src]

    def _segmax(msg):
        seg = jax.ops.segment_max(msg, dst, num_segments=NP)
        return jnp.where(jnp.isfinite(seg), seg, 0.0)

    t1 = _tables_call(x_pad, W1, b1r)
    e1 = jnp.concatenate(
        [_edge_pre(t1), jnp.zeros((EPAD - E, H), jnp.float32)], 0)
    ez = jnp.zeros((EPAD, H), jnp.float32)
    l2 = _mlp_call(jnp.concatenate([e1, ez], axis=1), W2, b2r)
    h1raw = _segmax(l2[:E, 0:H])

    t2 = _bn_tables_call(
        jnp.concatenate([h1raw, jnp.zeros((NP, H), jnp.float32)], 1),
        g1r, be1r, W3, b3r)
    e2 = jnp.concatenate(
        [_edge_pre(t2), jnp.zeros((EPAD - E, H), jnp.float32)], 0)
    l4 = _mlp_call(jnp.concatenate([e2, ez], axis=1), W4, b4r)
    h2raw = _segmax(l4[:E, 0:H])

    h2pad = jnp.concatenate([h2raw, jnp.zeros((NP, H), jnp.float32)], 1)
    h2bn, mean2, rstd2 = _bn_call(h2pad, g2r, be2r)
    x_graph = _pool_call(h2pad, batch2d, mean2, rstd2, g2r, be2r)
    return h2bn[:N] + keep, x_graph.reshape(G, 2 * H)
